# Initial kernel scaffold; baseline (speedup 1.0000x reference)
import jax
import jax.numpy as jnp
from jax.experimental import pallas as pl


def kernel(node_feature, edge_index, W1, b1, W2, b2, W3, b3):
    raise NotImplementedError("write your pallas kernel here")


# speedup vs baseline: 5.0996x; 5.0996x over previous
import functools

import jax
import jax.numpy as jnp
from jax import lax
from jax.experimental import pallas as pl
from jax.experimental.pallas import tpu as pltpu
from jax.experimental.pallas import tpu_sc as plsc

N = 10000
E = 320000
D = 128

NC = 2
NS = 16
NW = NC * NS

NPAD = 10240
ROWS_PER_TILE = NPAD // NS

EDGES_PER_W = E // NW
CHUNK = 80
NSTEPS = EDGES_PER_W // CHUNK
OUT_CHUNKS = ROWS_PER_TILE // CHUNK


def _sc_aggregate(node_feature, src, dst, zeros_init):
    mesh = plsc.VectorSubcoreMesh(
        core_axis_name="c", subcore_axis_name="s", num_cores=NC,
        num_subcores=NS)

    @functools.partial(
        pl.kernel,
        out_type=jax.ShapeDtypeStruct((NC * NPAD, D), jnp.float32),
        mesh=mesh,
        scratch_types=[
            pltpu.VMEM_SHARED((NPAD, D), jnp.float32),
            pltpu.VMEM((CHUNK,), jnp.int32),
            pltpu.VMEM((CHUNK,), jnp.int32),
            pltpu.VMEM((CHUNK, D), jnp.float32),
            pltpu.SemaphoreType.DMA,
        ],
    )
    def agg_kernel(nf_hbm, src_hbm, dst_hbm, zero_hbm, out_hbm,
                   acc_sh, src_v, dst_v, rows_v, sem):
        cid = lax.axis_index("c")
        sid = lax.axis_index("s")
        wid = sid * NC + cid

        row0 = pl.multiple_of(sid * ROWS_PER_TILE, 8)
        for j in range(OUT_CHUNKS):
            pltpu.sync_copy(zero_hbm, acc_sh.at[pl.ds(row0 + j * CHUNK, CHUNK)])
        plsc.subcore_barrier()

        ebase = pl.multiple_of(wid * EDGES_PER_W, 8)

        @pl.loop(0, NSTEPS)
        def _edge_step(i):
            base = pl.multiple_of(ebase + i * CHUNK, 8)
            pltpu.sync_copy(src_hbm.at[pl.ds(base, CHUNK)], src_v)
            pltpu.sync_copy(dst_hbm.at[pl.ds(base, CHUNK)], dst_v)
            pltpu.async_copy(nf_hbm.at[src_v], rows_v, sem).wait()
            pltpu.sync_copy(rows_v, acc_sh.at[dst_v], add=True)

        plsc.subcore_barrier()

        obase = cid * NPAD + sid * ROWS_PER_TILE
        for j in range(OUT_CHUNKS):
            pltpu.sync_copy(acc_sh.at[pl.ds(row0 + j * CHUNK, CHUNK)], rows_v)
            pltpu.sync_copy(rows_v, out_hbm.at[pl.ds(obase + j * CHUNK, CHUNK)])

    return agg_kernel(node_feature, src, dst, zeros_init)


def _mlp_block(p0, p1, nf, w1a, w1b, b1, w2, b2, w3, b3, out):
    agg = p0[...] + p1[...]
    h = agg @ w1a[...] + nf[...] @ w1b[...] + b1[...]
    h = jnp.maximum(h, 0.0)
    h = h @ w2[...] + b2[...]
    h = jnp.maximum(h, 0.0)
    out[...] = h @ w3[...] + b3[...]


def _tc_mlp(p0, p1, nf_pad, W1a, W1b, b1, W2, b2, W3, b3):
    BR = 1280
    grid = NPAD // BR
    row_spec = pl.BlockSpec((BR, D), lambda i: (i, 0))
    full2 = pl.BlockSpec((D, D), lambda i: (0, 0))
    bias = pl.BlockSpec((1, D), lambda i: (0, 0))
    return pl.pallas_call(
        _mlp_block,
        grid=(grid,),
        in_specs=[row_spec, row_spec, row_spec,
                  full2, full2, bias, full2, bias, full2, bias],
        out_specs=row_spec,
        out_shape=jax.ShapeDtypeStruct((NPAD, D), jnp.float32),
    )(p0, p1, nf_pad, W1a, W1b, b1, W2, b2, W3, b3)


@jax.jit
def kernel(node_feature, edge_index, W1, b1, W2, b2, W3, b3):
    src = edge_index[0]
    dst = edge_index[1]
    zeros_init = jnp.zeros((CHUNK, D), jnp.float32)

    partials = _sc_aggregate(node_feature, src, dst, zeros_init)
    p0 = partials[:NPAD]
    p1 = partials[NPAD:]

    nf_pad = jnp.pad(node_feature, ((0, NPAD - N), (0, 0)))
    out = _tc_mlp(p0, p1, nf_pad,
                  W1[:D], W1[D:], b1.reshape(1, D),
                  W2, b2.reshape(1, D), W3, b3.reshape(1, D))
    return out[:N]
